# baseline (speedup 1.0000x reference)
import functools

import jax
import jax.numpy as jnp
from jax.experimental import pallas as pl
from jax.experimental.pallas import tpu as pltpu

_GROUP = 4


def _mlp_body(n_slabs, x_ref, valid_ref, w1_ref, b1_ref, w2_ref, b2_ref, out_ref):
    sb = x_ref.shape[0]
    x = x_ref[...].reshape(sb, -1)
    x = jnp.maximum(x, 0.0).astype(jnp.bfloat16)
    acc = None
    for g in range(n_slabs):
        xg = x[:, 128 * g:128 * (g + 1)]
        hg = jnp.dot(xg, w1_ref[g], preferred_element_type=jnp.float32)
        hg = jnp.maximum(hg + b1_ref[g][None, :], 0.0).astype(jnp.bfloat16)
        cg = jnp.dot(hg, w2_ref[g], preferred_element_type=jnp.float32)
        acc = cg if acc is None else acc + cg
    out = acc + b2_ref[...]
    out_ref[...] = jnp.where(valid_ref[...] > 0, out, 0.0)


@functools.partial(jax.jit, static_argnames=("block_rows",))
def _run(h, valid, W1, b1, W2, b2, block_rows=1024):
    B, N, C = h.shape
    Wh = W1.shape[2]
    G = _GROUP
    S = N // G
    lanes = G * C

    eye_g = jnp.eye(G, dtype=W1.dtype)
    W1s = W1.reshape(S, G, C, Wh)
    W1bd = (W1s[:, :, :, None, :] * eye_g[None, :, None, :, None]).reshape(
        S, G * C, G * Wh).astype(jnp.bfloat16)
    b1p = b1.reshape(S, G * Wh)

    W2s = W2[:, :, 0].reshape(S, G, Wh)
    eye_n = jnp.eye(N, dtype=W2.dtype).reshape(S, G, 1, N)
    W2p = (W2s[:, :, :, None] * eye_n).reshape(S, G * Wh, N).astype(jnp.bfloat16)
    b2p = b2[:, 0][None, :]

    grid = (B // block_rows,)
    out = pl.pallas_call(
        functools.partial(_mlp_body, S),
        grid=grid,
        in_specs=[
            pl.BlockSpec((block_rows, N, C), lambda i: (i, 0, 0)),
            pl.BlockSpec((block_rows, N), lambda i: (i, 0)),
            pl.BlockSpec((S, lanes, lanes), lambda i: (0, 0, 0)),
            pl.BlockSpec((S, lanes), lambda i: (0, 0)),
            pl.BlockSpec((S, lanes, N), lambda i: (0, 0, 0)),
            pl.BlockSpec((1, N), lambda i: (0, 0)),
        ],
        out_specs=pl.BlockSpec((block_rows, N), lambda i: (i, 0)),
        out_shape=jax.ShapeDtypeStruct((B, N), jnp.float32),
        compiler_params=pltpu.CompilerParams(
            dimension_semantics=("arbitrary",),
        ),
    )(h, valid, W1bd, b1p, W2p, b2p)
    return out.reshape(B, N, 1)


def kernel(h, valid, W1, b1, W2, b2):
    return _run(h, valid, W1, b1, W2, b2)


# speedup vs baseline: 7.0240x; 7.0240x over previous
import functools

import jax
import jax.numpy as jnp
from jax.experimental import pallas as pl
from jax.experimental.pallas import tpu as pltpu

_GROUP = 4


def _mlp_body(n_slabs, x_ref, valid_ref, w1_ref, b1_ref, w2_ref, b2_ref, out_ref):
    x = jnp.maximum(x_ref[...], 0.0).astype(jnp.bfloat16)
    acc = None
    for g in range(n_slabs):
        xg = x[128 * g:128 * (g + 1), :]
        hg = jnp.dot(w1_ref[g], xg, preferred_element_type=jnp.float32)
        hg = jnp.maximum(hg + b1_ref[g][:, None], 0.0).astype(jnp.bfloat16)
        cg = jnp.dot(w2_ref[g], hg, preferred_element_type=jnp.float32)
        acc = cg if acc is None else acc + cg
    out = acc + b2_ref[...]
    out_ref[...] = jnp.where(valid_ref[...] > 0, out, 0.0)


@functools.partial(jax.jit, static_argnames=("block_lanes",))
def _run(h, valid, W1, b1, W2, b2, block_lanes=2048):
    B, N, C = h.shape
    Wh = W1.shape[2]
    G = _GROUP
    S = N // G
    R = G * C

    xT = h.transpose(1, 2, 0).reshape(N * C, B)
    validT = valid.transpose(1, 0)

    eye_g = jnp.eye(G, dtype=W1.dtype)
    W1Ts = W1.transpose(0, 2, 1).reshape(S, G, Wh, C)
    W1bdT = (W1Ts[:, :, :, None, :] * eye_g[None, :, None, :, None]).reshape(
        S, G * Wh, G * C).astype(jnp.bfloat16)
    b1bd = b1.reshape(S, G * Wh)

    ind = jnp.transpose(jnp.eye(N, dtype=W2.dtype).reshape(N, S, G), (1, 0, 2))
    W2row = (ind[:, :, :, None] * W2[None, :, None, :, 0]).reshape(
        S, N, G * Wh).astype(jnp.bfloat16)

    grid = (B // block_lanes,)
    outT = pl.pallas_call(
        functools.partial(_mlp_body, S),
        grid=grid,
        in_specs=[
            pl.BlockSpec((N * C, block_lanes), lambda j: (0, j)),
            pl.BlockSpec((N, block_lanes), lambda j: (0, j)),
            pl.BlockSpec((S, R, R), lambda j: (0, 0, 0)),
            pl.BlockSpec((S, R), lambda j: (0, 0)),
            pl.BlockSpec((S, N, R), lambda j: (0, 0, 0)),
            pl.BlockSpec((N, 1), lambda j: (0, 0)),
        ],
        out_specs=pl.BlockSpec((N, block_lanes), lambda j: (0, j)),
        out_shape=jax.ShapeDtypeStruct((N, B), jnp.float32),
        compiler_params=pltpu.CompilerParams(
            dimension_semantics=("arbitrary",),
        ),
    )(xT, validT, W1bdT, b1bd, W2row, b2)
    return outT.transpose(1, 0)[:, :, None]


def kernel(h, valid, W1, b1, W2, b2):
    return _run(h, valid, W1, b1, W2, b2)


# speedup vs baseline: 7.6232x; 1.0853x over previous
import functools

import jax
import jax.numpy as jnp
from jax.experimental import pallas as pl
from jax.experimental.pallas import tpu as pltpu


def _mlp_body(n_nodes, x_ref, valid_ref, w1_ref, b1_ref, w2_ref, b2_ref, out_ref):
    x = jnp.maximum(x_ref[...], 0.0).astype(jnp.bfloat16)
    hs = []
    for n in range(n_nodes):
        xn = x[32 * n:32 * (n + 1), :]
        hn = jnp.dot(w1_ref[n], xn, preferred_element_type=jnp.float32)
        hn = jnp.maximum(hn + b1_ref[n][:, None], 0.0).astype(jnp.bfloat16)
        hs.append(hn)
    hid = jnp.concatenate(hs, axis=0)
    out = jnp.dot(w2_ref[...], hid, preferred_element_type=jnp.float32)
    out = out + b2_ref[...]
    out_ref[...] = jnp.where(valid_ref[...] > 0, out, 0.0)


@functools.partial(jax.jit, static_argnames=("block_lanes",))
def _run(h, valid, W1, b1, W2, b2, block_lanes=2048):
    B, N, C = h.shape
    Wh = W1.shape[2]

    xT = h.transpose(1, 2, 0).reshape(N * C, B)
    validT = valid.transpose(1, 0)

    W1T = W1.transpose(0, 2, 1).astype(jnp.bfloat16)

    eye_n = jnp.eye(N, dtype=W2.dtype)
    W2row = (eye_n[:, :, None] * W2[None, :, :, 0]).reshape(N, N * Wh).astype(jnp.bfloat16)

    grid = (B // block_lanes,)
    outT = pl.pallas_call(
        functools.partial(_mlp_body, N),
        grid=grid,
        in_specs=[
            pl.BlockSpec((N * C, block_lanes), lambda j: (0, j)),
            pl.BlockSpec((N, block_lanes), lambda j: (0, j)),
            pl.BlockSpec((N, Wh, C), lambda j: (0, 0, 0)),
            pl.BlockSpec((N, Wh), lambda j: (0, 0)),
            pl.BlockSpec((N, N * Wh), lambda j: (0, 0)),
            pl.BlockSpec((N, 1), lambda j: (0, 0)),
        ],
        out_specs=pl.BlockSpec((N, block_lanes), lambda j: (0, j)),
        out_shape=jax.ShapeDtypeStruct((N, B), jnp.float32),
        compiler_params=pltpu.CompilerParams(
            dimension_semantics=("arbitrary",),
        ),
    )(xT, validT, W1T, b1, W2row, b2)
    return outT.transpose(1, 0)[:, :, None]


def kernel(h, valid, W1, b1, W2, b2):
    return _run(h, valid, W1, b1, W2, b2)


# speedup vs baseline: 9.6105x; 1.2607x over previous
import functools

import jax
import jax.numpy as jnp
from jax.experimental import pallas as pl
from jax.experimental.pallas import tpu as pltpu


def _mlp_body(n_nodes, x_ref, valid_ref, w1_ref, b1_ref, w2_ref, b2_ref, out_ref):
    x = jnp.maximum(x_ref[...], 0.0).astype(jnp.bfloat16)
    hs = []
    for n in range(n_nodes):
        xn = x[32 * n:32 * (n + 1), :]
        hn = jnp.dot(w1_ref[n], xn, preferred_element_type=jnp.float32)
        hn = jnp.maximum(hn + b1_ref[n][:, None], 0.0).astype(jnp.bfloat16)
        hs.append(hn)
    hid = jnp.concatenate(hs, axis=0)
    out = jnp.dot(w2_ref[...], hid, preferred_element_type=jnp.float32)
    out = out + b2_ref[...]
    out = jnp.where(valid_ref[...] > 0, out, 0.0)
    out_ref[...] = out.reshape(out_ref.shape)


@functools.partial(jax.jit, static_argnames=("block_lanes",))
def _run(h, valid, W1, b1, W2, b2, block_lanes=2048):
    B, N, C = h.shape
    Wh = W1.shape[2]

    xT = h.transpose(1, 2, 0).reshape(N * C, B)
    validT = valid.transpose(1, 0)

    W1T = W1.transpose(0, 2, 1).astype(jnp.bfloat16)

    eye_n = jnp.eye(N, dtype=W2.dtype)
    W2row = (eye_n[:, :, None] * W2[None, :, :, 0]).reshape(N, N * Wh).astype(jnp.bfloat16)

    grid = (B // block_lanes,)
    outT = pl.pallas_call(
        functools.partial(_mlp_body, N),
        grid=grid,
        in_specs=[
            pl.BlockSpec((N * C, block_lanes), lambda j: (0, j)),
            pl.BlockSpec((N, block_lanes), lambda j: (0, j)),
            pl.BlockSpec((N, Wh, C), lambda j: (0, 0, 0)),
            pl.BlockSpec((N, Wh), lambda j: (0, 0)),
            pl.BlockSpec((N, N * Wh), lambda j: (0, 0)),
            pl.BlockSpec((N, 1), lambda j: (0, 0)),
        ],
        out_specs=pl.BlockSpec((N, block_lanes // 128, 128), lambda j: (0, j, 0)),
        out_shape=jax.ShapeDtypeStruct((N, B // 128, 128), jnp.float32),
        compiler_params=pltpu.CompilerParams(
            dimension_semantics=("arbitrary",),
        ),
    )(xT, validT, W1T, b1, W2row, b2)
    return outT.transpose(1, 2, 0).reshape(B, N)[:, :, None]


def kernel(h, valid, W1, b1, W2, b2):
    return _run(h, valid, W1, b1, W2, b2)


# speedup vs baseline: 10.5153x; 1.0942x over previous
import functools

import jax
import jax.numpy as jnp
from jax.experimental import pallas as pl
from jax.experimental.pallas import tpu as pltpu


def _mlp_body(n_nodes, x_ref, valid_ref, w1_ref, b1_ref, w2_ref, b2_ref, out_ref):
    x = jnp.maximum(x_ref[...], 0.0).astype(jnp.bfloat16)
    hs = []
    for n in range(n_nodes):
        xn = x[32 * n:32 * (n + 1), :]
        hn = jnp.dot(w1_ref[n], xn, preferred_element_type=jnp.float32)
        hn = jnp.maximum(hn + b1_ref[n][:, None], 0.0).astype(jnp.bfloat16)
        hs.append(hn)
    hid = jnp.concatenate(hs, axis=0)
    out = jnp.dot(w2_ref[...], hid, preferred_element_type=jnp.float32)
    out = out + b2_ref[...]
    out = jnp.where(valid_ref[...] > 0, out, 0.0)
    out_ref[...] = out.reshape(out_ref.shape)


@functools.partial(jax.jit, static_argnames=("block_lanes",))
def _run(h, valid, W1, b1, W2, b2, block_lanes=4096):
    B, N, C = h.shape
    Wh = W1.shape[2]

    xT = h.transpose(1, 2, 0).reshape(N * C, B)
    validT = valid.transpose(1, 0)

    W1T = W1.transpose(0, 2, 1).astype(jnp.bfloat16)

    eye_n = jnp.eye(N, dtype=W2.dtype)
    W2row = (eye_n[:, :, None] * W2[None, :, :, 0]).reshape(N, N * Wh).astype(jnp.bfloat16)

    grid = (B // block_lanes,)
    outT = pl.pallas_call(
        functools.partial(_mlp_body, N),
        grid=grid,
        in_specs=[
            pl.BlockSpec((N * C, block_lanes), lambda j: (0, j)),
            pl.BlockSpec((N, block_lanes), lambda j: (0, j)),
            pl.BlockSpec((N, Wh, C), lambda j: (0, 0, 0)),
            pl.BlockSpec((N, Wh), lambda j: (0, 0)),
            pl.BlockSpec((N, N * Wh), lambda j: (0, 0)),
            pl.BlockSpec((N, 1), lambda j: (0, 0)),
        ],
        out_specs=pl.BlockSpec((N, block_lanes // 128, 128), lambda j: (0, j, 0)),
        out_shape=jax.ShapeDtypeStruct((N, B // 128, 128), jnp.float32),
        compiler_params=pltpu.CompilerParams(
            dimension_semantics=("arbitrary",),
        ),
    )(xT, validT, W1T, b1, W2row, b2)
    return outT.transpose(1, 2, 0).reshape(B, N)[:, :, None]


def kernel(h, valid, W1, b1, W2, b2):
    return _run(h, valid, W1, b1, W2, b2)
